# baseline (speedup 1.0000x reference)
import functools

import jax
import jax.numpy as jnp
from jax import lax
from jax.experimental import pallas as pl
from jax.experimental.pallas import tpu as pltpu
from jax.experimental.pallas import tpu_sc as plsc

NC = 2
NS = 16
NW = NC * NS
LANES = 16
CHUNK = 128
H = 64
DEPTH = 8
NROW = 1_000_000


def _build(B):
    bpw = B // NW
    nch = bpw // CHUNK

    mesh = plsc.VectorSubcoreMesh(core_axis_name="c", subcore_axis_name="s")

    @functools.partial(
        pl.kernel,
        out_type=jax.ShapeDtypeStruct((B,), jnp.float32),
        mesh=mesh,
        compiler_params=pltpu.CompilerParams(
            needs_layout_passes=False, use_tc_tiling_on_sc=False),
        scratch_types=[
            pltpu.VMEM((nch, CHUNK), jnp.int32),
            pltpu.VMEM((nch, CHUNK), jnp.int32),
            pltpu.VMEM((H, CHUNK), jnp.int32),
            pltpu.VMEM((H, CHUNK), jnp.int32),
            pltpu.VMEM((H * CHUNK,), jnp.float32),
            pltpu.VMEM((H * CHUNK,), jnp.float32),
            pltpu.VMEM((bpw,), jnp.float32),
            pltpu.VMEM((bpw,), jnp.float32),
            pltpu.VMEM((bpw,), jnp.float32),
            pltpu.VMEM((LANES,), jnp.float32),
            pltpu.SemaphoreType.DMA,
        ],
    )
    def mf(user_hbm, item_hbm, uw_hbm, iw_hbm, ub_hbm, ib_hbm, bias_hbm,
           out_hbm, uid_v, iid_v, uhx_v, ihx_v, uval_v, ival_v,
           ubr_v, ibr_v, out_v, bias_v, sem):
        wid = lax.axis_index("s") * NC + lax.axis_index("c")
        base = wid * bpw

        pltpu.sync_copy(user_hbm.at[wid], uid_v)
        pltpu.sync_copy(item_hbm.at[wid], iid_v)
        pltpu.sync_copy(bias_hbm, bias_v)

        bias_copies = []
        for c in range(nch):
            sl = pl.ds(c * CHUNK, CHUNK)
            bias_copies.append(pltpu.make_async_copy(
                ub_hbm.at[uid_v.at[c]], ubr_v.at[sl], sem))
            bias_copies.append(pltpu.make_async_copy(
                ib_hbm.at[iid_v.at[c]], ibr_v.at[sl], sem))
        for cp in bias_copies:
            cp.start()

        bias_vec = bias_v[...]

        for c in range(nch):
            def build_body(h, carry, _c=c):
                coff = h * NROW
                for g in range(CHUNK // LANES):
                    sl = pl.ds(g * LANES, LANES)
                    uhx_v[h, sl] = uid_v[_c, sl] + coff
                    ihx_v[h, sl] = iid_v[_c, sl] + coff
                return carry

            lax.fori_loop(0, H, build_body, 0)

            def fire_one(h):
                dst = pl.ds(h * CHUNK, CHUNK)
                pltpu.make_async_copy(
                    uw_hbm.at[uhx_v.at[h]], uval_v.at[dst], sem).start()
                pltpu.make_async_copy(
                    iw_hbm.at[ihx_v.at[h]], ival_v.at[dst], sem).start()

            def wait_one(h):
                dst = pl.ds(h * CHUNK, CHUNK)
                pltpu.make_async_copy(
                    uw_hbm.at[uhx_v.at[h]], uval_v.at[dst], sem).wait()
                pltpu.make_async_copy(
                    iw_hbm.at[ihx_v.at[h]], ival_v.at[dst], sem).wait()

            def fire_body(h, carry):
                fire_one(h)

                @pl.when(h >= DEPTH)
                def _():
                    wait_one(h - DEPTH)
                return carry

            lax.fori_loop(0, H, fire_body, 0)

            def drain_body(h, carry):
                wait_one(h)
                return carry

            lax.fori_loop(H - DEPTH, H, drain_body, 0)
            if c == 0:
                for cp in bias_copies:
                    cp.wait()

            for g in range(CHUNK // LANES):
                o = c * CHUNK + g * LANES
                ubv = ubr_v[pl.ds(o, LANES)]
                ibv = ibr_v[pl.ds(o, LANES)]

                def h_body(h, acc, _g=g):
                    u16 = uval_v[pl.ds(h * CHUNK + _g * LANES, LANES)]
                    i16 = ival_v[pl.ds(h * CHUNK + _g * LANES, LANES)]
                    return acc + (u16 + ubv) * (i16 + ibv)

                acc = lax.fori_loop(
                    0, H, h_body, jnp.zeros((LANES,), jnp.float32))
                out_v[pl.ds(o, LANES)] = acc + bias_vec

        pltpu.sync_copy(out_v, out_hbm.at[pl.ds(base, bpw)])

    return mf


def kernel(user, item, user_weight, item_weight, user_bias, item_bias, bias):
    B = user.shape[0]
    user_r = user.reshape(NW, B // NW // CHUNK, CHUNK)
    item_r = item.reshape(NW, B // NW // CHUNK, CHUNK)
    uw_flat = user_weight.T.reshape(-1)
    iw_flat = item_weight.T.reshape(-1)
    ub = user_bias.reshape(-1)
    ib = item_bias.reshape(-1)
    bias16 = jnp.broadcast_to(bias, (LANES,)).astype(jnp.float32)
    mf = _build(B)
    return mf(user_r, item_r, uw_flat, iw_flat, ub, ib, bias16)


# speedup vs baseline: 17.9446x; 17.9446x over previous
import functools

import jax
import jax.numpy as jnp
from jax import lax
from jax.experimental import pallas as pl
from jax.experimental.pallas import tpu as pltpu
from jax.experimental.pallas import tpu_sc as plsc

NC = 2
NS = 16
NW = NC * NS
LANES = 16
CHUNK = 128
H = 64
DEPTH = 8
NROW = 1_000_000
PB = 1 << 17
NJ = 8
A_STRIDE = 8 * 8 * PB
J_STRIDE = 8 * PB
FLAT = 8 * A_STRIDE


def _tc_detile(wt3):
    def body(in_ref, out_ref):
        out_ref[...] = in_ref[0].reshape(J_STRIDE // 128, 128)

    return pl.pallas_call(
        body,
        grid=(8, NJ),
        in_specs=[pl.BlockSpec((1, 8, PB), lambda a, j: (a, 0, j))],
        out_specs=pl.BlockSpec(
            (J_STRIDE // 128, 128), lambda a, j: (a * NJ + j, 0)),
        out_shape=jax.ShapeDtypeStruct((FLAT // 128, 128), jnp.float32),
    )(wt3)


def _build_sc(B):
    bpw = B // NW
    nch = bpw // CHUNK

    mesh = plsc.VectorSubcoreMesh(core_axis_name="c", subcore_axis_name="s")

    @functools.partial(
        pl.kernel,
        out_type=jax.ShapeDtypeStruct((B,), jnp.float32),
        mesh=mesh,
        compiler_params=pltpu.CompilerParams(
            needs_layout_passes=False, use_tc_tiling_on_sc=False),
        scratch_types=[
            pltpu.VMEM((nch, CHUNK), jnp.int32),
            pltpu.VMEM((nch, CHUNK), jnp.int32),
            pltpu.VMEM((H, CHUNK), jnp.int32),
            pltpu.VMEM((H, CHUNK), jnp.int32),
            pltpu.VMEM((H * CHUNK,), jnp.float32),
            pltpu.VMEM((H * CHUNK,), jnp.float32),
            pltpu.VMEM((bpw,), jnp.float32),
            pltpu.VMEM((bpw,), jnp.float32),
            pltpu.VMEM((bpw,), jnp.float32),
            pltpu.VMEM((LANES,), jnp.float32),
            pltpu.SemaphoreType.DMA,
        ],
    )
    def mf(user_hbm, item_hbm, uw_hbm, iw_hbm, ub_hbm, ib_hbm, bias_hbm,
           out_hbm, uid_v, iid_v, uhx_v, ihx_v, uval_v, ival_v,
           ubr_v, ibr_v, out_v, bias_v, sem):
        wid = lax.axis_index("s") * NC + lax.axis_index("c")
        base = wid * bpw

        pltpu.sync_copy(user_hbm.at[wid], uid_v)
        pltpu.sync_copy(item_hbm.at[wid], iid_v)
        pltpu.sync_copy(bias_hbm, bias_v)

        bias_copies = []
        for c in range(nch):
            sl = pl.ds(c * CHUNK, CHUNK)
            bias_copies.append(pltpu.make_async_copy(
                ub_hbm.at[uid_v.at[c]], ubr_v.at[sl], sem))
            bias_copies.append(pltpu.make_async_copy(
                ib_hbm.at[iid_v.at[c]], ibr_v.at[sl], sem))
        for cp in bias_copies:
            cp.start()

        bias_vec = bias_v[...]

        for c in range(nch):
            def build_body(h, carry, _c=c):
                coff = (h >> 3) * A_STRIDE + (h & 7) * PB
                for g in range(CHUNK // LANES):
                    sl = pl.ds(g * LANES, LANES)
                    r = uid_v[_c, sl]
                    uhx_v[h, sl] = (
                        (r >> 17) * J_STRIDE + (r & (PB - 1)) + coff)
                    r = iid_v[_c, sl]
                    ihx_v[h, sl] = (
                        (r >> 17) * J_STRIDE + (r & (PB - 1)) + coff)
                return carry

            lax.fori_loop(0, H, build_body, 0)

            def fire_one(h):
                dst = pl.ds(h * CHUNK, CHUNK)
                pltpu.make_async_copy(
                    uw_hbm.at[uhx_v.at[h]], uval_v.at[dst], sem).start()
                pltpu.make_async_copy(
                    iw_hbm.at[ihx_v.at[h]], ival_v.at[dst], sem).start()

            def wait_one(h):
                dst = pl.ds(h * CHUNK, CHUNK)
                pltpu.make_async_copy(
                    uw_hbm.at[uhx_v.at[h]], uval_v.at[dst], sem).wait()
                pltpu.make_async_copy(
                    iw_hbm.at[ihx_v.at[h]], ival_v.at[dst], sem).wait()

            def fire_body(h, carry):
                fire_one(h)

                @pl.when(h >= DEPTH)
                def _():
                    wait_one(h - DEPTH)
                return carry

            lax.fori_loop(0, H, fire_body, 0)

            def drain_body(h, carry):
                wait_one(h)
                return carry

            lax.fori_loop(H - DEPTH, H, drain_body, 0)
            if c == 0:
                for cp in bias_copies:
                    cp.wait()

            for g in range(CHUNK // LANES):
                o = c * CHUNK + g * LANES
                ubv = ubr_v[pl.ds(o, LANES)]
                ibv = ibr_v[pl.ds(o, LANES)]

                def h_body(h, acc, _g=g):
                    u16 = uval_v[pl.ds(h * CHUNK + _g * LANES, LANES)]
                    i16 = ival_v[pl.ds(h * CHUNK + _g * LANES, LANES)]
                    return acc + (u16 + ubv) * (i16 + ibv)

                acc = lax.fori_loop(
                    0, H, h_body, jnp.zeros((LANES,), jnp.float32))
                out_v[pl.ds(o, LANES)] = acc + bias_vec

        pltpu.sync_copy(out_v, out_hbm.at[pl.ds(base, bpw)])

    return mf


def kernel(user, item, user_weight, item_weight, user_bias, item_bias, bias):
    B = user.shape[0]
    user_r = user.reshape(NW, B // NW // CHUNK, CHUNK)
    item_r = item.reshape(NW, B // NW // CHUNK, CHUNK)
    uw_flat = _tc_detile(user_weight.T.reshape(8, 8, NROW)).reshape(-1)
    iw_flat = _tc_detile(item_weight.T.reshape(8, 8, NROW)).reshape(-1)
    ub = user_bias.reshape(-1)
    ib = item_bias.reshape(-1)
    bias16 = jnp.broadcast_to(bias, (LANES,)).astype(jnp.float32)
    mf = _build_sc(B)
    return mf(user_r, item_r, uw_flat, iw_flat, ub, ib, bias16)
